# Initial kernel scaffold; baseline (speedup 1.0000x reference)
import jax
import jax.numpy as jnp
from jax.experimental import pallas as pl


def kernel(K_h, Q_h, P_e, edge_index, W1, b1, W2, b2):
    raise NotImplementedError("write your pallas kernel here")


# speedup vs baseline: 1.0095x; 1.0095x over previous
import functools

import jax
import jax.numpy as jnp
from jax import lax
from jax.experimental import pallas as pl
from jax.experimental.pallas import tpu as pltpu
from jax.experimental.pallas import tpu_sc as plsc

D = 128
LANES = 16
NC, NS = 2, 16
NW = NC * NS
CHUNK = 128


def _tables_body(k_ref, p_ref, q_ref, w1_ref, b1_ref, a_ref, b_ref):
    x = k_ref[...] + p_ref[...]
    w1 = w1_ref[...]
    a_ref[...] = jnp.dot(x, w1, preferred_element_type=jnp.float32) + b1_ref[...]
    b_ref[...] = jnp.dot(q_ref[...], w1, preferred_element_type=jnp.float32)


def _node_tables(K_h, Q_h, P_e, W1, b1):
    n = K_h.shape[0]
    blk = 1000
    grid = (n // blk,)
    row_spec = pl.BlockSpec((blk, D), lambda i: (i, 0))
    return pl.pallas_call(
        _tables_body,
        grid=grid,
        in_specs=[row_spec, row_spec, row_spec,
                  pl.BlockSpec((D, D), lambda i: (0, 0)),
                  pl.BlockSpec((1, D), lambda i: (0, 0))],
        out_specs=[row_spec, row_spec],
        out_shape=[jax.ShapeDtypeStruct((n, D), jnp.float32),
                   jax.ShapeDtypeStruct((n, D), jnp.float32)],
    )(K_h, P_e, Q_h, W1, b1.reshape(1, D))


def _edge_scores(A, B, src, dst, w2_flat, b2_vec, e_pad):
    per_w = e_pad // NW
    n_chunks = per_w // CHUNK
    mesh = plsc.VectorSubcoreMesh(core_axis_name="c", subcore_axis_name="s",
                                  num_cores=NC, num_subcores=NS)

    @functools.partial(
        pl.kernel,
        out_type=jax.ShapeDtypeStruct((e_pad,), jnp.float32),
        mesh=mesh,
        compiler_params=pltpu.CompilerParams(needs_layout_passes=False),
        scratch_types=[
            pltpu.VMEM((CHUNK,), jnp.int32),
            pltpu.VMEM((CHUNK,), jnp.int32),
            pltpu.VMEM((CHUNK, D), jnp.float32),
            pltpu.VMEM((CHUNK, D), jnp.float32),
            pltpu.VMEM((D,), jnp.float32),
            pltpu.VMEM((LANES,), jnp.float32),
            pltpu.VMEM((CHUNK,), jnp.float32),
            pltpu.SemaphoreType.DMA,
            pltpu.SemaphoreType.DMA,
        ],
    )
    def k(a_hbm, b_hbm, src_hbm, dst_hbm, w2_hbm, b2_hbm, out_hbm,
          idx_s, idx_d, rows_a, rows_b, w2_v, b2_v, out_v, sem_a, sem_b):
        wid = lax.axis_index("s") * NC + lax.axis_index("c")
        base_w = wid * per_w
        pltpu.sync_copy(w2_hbm, w2_v)
        pltpu.sync_copy(b2_hbm, b2_v)
        iot = lax.iota(jnp.int32, LANES)
        n_groups = CHUNK // LANES

        def chunk_body(c, carry):
            base = base_w + c * CHUNK
            pltpu.sync_copy(src_hbm.at[pl.ds(base, CHUNK)], idx_s)
            pltpu.sync_copy(dst_hbm.at[pl.ds(base, CHUNK)], idx_d)
            cp_a = pltpu.async_copy(a_hbm.at[idx_s], rows_a, sem_a)
            cp_b = pltpu.async_copy(b_hbm.at[idx_d], rows_b, sem_b)
            cp_a.wait()
            cp_b.wait()
            b2row = b2_v[...]

            def fc_body(fc, accs):
                fbase = fc * LANES
                w2c = w2_v[pl.ds(fbase, LANES)]
                out = list(accs)
                for j in range(LANES):
                    w2vec = w2c.at[jnp.full((LANES,), j, jnp.int32)].get(
                        mode="promise_in_bounds")
                    fvec = jnp.zeros((LANES,), jnp.int32) + (fbase + j)
                    for g in range(n_groups):
                        eidx = iot + (LANES * g)
                        va = plsc.load_gather(rows_a, [eidx, fvec])
                        vb = plsc.load_gather(rows_b, [eidx, fvec])
                        out[g] = out[g] + jnp.maximum(va - vb, 0.0) * w2vec
                return tuple(out)

            accs = lax.fori_loop(0, D // LANES, fc_body,
                                 tuple(b2row for _ in range(n_groups)))
            for g in range(n_groups):
                out_v[pl.ds(LANES * g, LANES)] = accs[g]
            pltpu.sync_copy(out_v, out_hbm.at[pl.ds(base, CHUNK)])
            return carry

        lax.fori_loop(0, n_chunks, chunk_body, 0)

    return k(A, B, src, dst, w2_flat, b2_vec)


def kernel(K_h, Q_h, P_e, edge_index, W1, b1, W2, b2):
    n_edges = edge_index.shape[1]
    A, B = _node_tables(K_h, Q_h, P_e, W1, b1)

    grain = NW * CHUNK
    e_pad = ((n_edges + grain - 1) // grain) * grain
    pad = e_pad - n_edges
    src = jnp.concatenate([edge_index[0], jnp.zeros((pad,), jnp.int32)])
    dst = jnp.concatenate([edge_index[1], jnp.zeros((pad,), jnp.int32)])
    w2_flat = W2.reshape(D)
    b2_vec = jnp.broadcast_to(b2.reshape(1), (LANES,)).astype(jnp.float32)

    scores = _edge_scores(A, B, src, dst, w2_flat, b2_vec, e_pad)
    return scores[:n_edges].reshape(n_edges, 1)


# speedup vs baseline: 2.3275x; 2.3055x over previous
import functools

import jax
import jax.numpy as jnp
from jax import lax
from jax.experimental import pallas as pl
from jax.experimental.pallas import tpu as pltpu
from jax.experimental.pallas import tpu_sc as plsc

D = 128
NC, NS = 2, 16
NW = NC * NS
CHUNK = 128


def _tables_body(k_ref, p_ref, q_ref, w1_ref, b1_ref, a_ref, b_ref):
    x = k_ref[...] + p_ref[...]
    w1 = w1_ref[...]
    a_ref[...] = jnp.dot(x, w1, preferred_element_type=jnp.float32) + b1_ref[...]
    b_ref[...] = jnp.dot(q_ref[...], w1, preferred_element_type=jnp.float32)


def _node_tables(K_h, Q_h, P_e, W1, b1):
    n = K_h.shape[0]
    blk = 1000
    row_spec = pl.BlockSpec((blk, D), lambda i: (i, 0))
    return pl.pallas_call(
        _tables_body,
        grid=(n // blk,),
        in_specs=[row_spec, row_spec, row_spec,
                  pl.BlockSpec((D, D), lambda i: (0, 0)),
                  pl.BlockSpec((1, D), lambda i: (0, 0))],
        out_specs=[row_spec, row_spec],
        out_shape=[jax.ShapeDtypeStruct((n, D), jnp.float32),
                   jax.ShapeDtypeStruct((n, D), jnp.float32)],
    )(K_h, P_e, Q_h, W1, b1.reshape(1, D))


def _gather_rows(A, B, src, dst, e_pad):
    per_w = e_pad // NW
    n_chunks = per_w // CHUNK
    mesh = plsc.VectorSubcoreMesh(core_axis_name="c", subcore_axis_name="s",
                                  num_cores=NC, num_subcores=NS)

    @functools.partial(
        pl.kernel,
        out_type=[jax.ShapeDtypeStruct((e_pad, D), jnp.float32),
                  jax.ShapeDtypeStruct((e_pad, D), jnp.float32)],
        mesh=mesh,
        compiler_params=pltpu.CompilerParams(needs_layout_passes=False),
        scratch_types=[
            pltpu.VMEM((CHUNK,), jnp.int32),
            pltpu.VMEM((CHUNK,), jnp.int32),
            pltpu.VMEM((CHUNK, D), jnp.float32),
            pltpu.VMEM((CHUNK, D), jnp.float32),
            pltpu.SemaphoreType.DMA,
            pltpu.SemaphoreType.DMA,
        ],
    )
    def k(a_hbm, b_hbm, src_hbm, dst_hbm, ga_hbm, gb_hbm,
          idx_s, idx_d, buf_a, buf_b, sem_a, sem_b):
        wid = lax.axis_index("s") * NC + lax.axis_index("c")
        base_w = wid * per_w

        def chunk_body(c, carry):
            base = base_w + c * CHUNK
            pltpu.sync_copy(src_hbm.at[pl.ds(base, CHUNK)], idx_s)
            pltpu.sync_copy(dst_hbm.at[pl.ds(base, CHUNK)], idx_d)
            cp_a = pltpu.async_copy(a_hbm.at[idx_s], buf_a, sem_a)
            cp_b = pltpu.async_copy(b_hbm.at[idx_d], buf_b, sem_b)
            cp_a.wait()
            cp_b.wait()
            pltpu.sync_copy(buf_a, ga_hbm.at[pl.ds(base, CHUNK)])
            pltpu.sync_copy(buf_b, gb_hbm.at[pl.ds(base, CHUNK)])
            return carry

        lax.fori_loop(0, n_chunks, chunk_body, 0)

    return k(A, B, src, dst)


def _score_body(ga_ref, gb_ref, w2_ref, b2_ref, out_ref):
    h = jnp.maximum(ga_ref[...] - gb_ref[...], 0.0)
    out_ref[...] = (jnp.dot(h, w2_ref[...], preferred_element_type=jnp.float32)
                    + b2_ref[...])


def _edge_scores(GA, GB, W2, b2, e_pad):
    blk = 2048
    row_spec = pl.BlockSpec((blk, D), lambda i: (i, 0))
    return pl.pallas_call(
        _score_body,
        grid=(e_pad // blk,),
        in_specs=[row_spec, row_spec,
                  pl.BlockSpec((D, 1), lambda i: (0, 0)),
                  pl.BlockSpec((1, 1), lambda i: (0, 0))],
        out_specs=pl.BlockSpec((blk, 1), lambda i: (i, 0)),
        out_shape=jax.ShapeDtypeStruct((e_pad, 1), jnp.float32),
    )(GA, GB, W2, b2.reshape(1, 1))


def kernel(K_h, Q_h, P_e, edge_index, W1, b1, W2, b2):
    n_edges = edge_index.shape[1]
    A, B = _node_tables(K_h, Q_h, P_e, W1, b1)

    grain = NW * CHUNK
    e_pad = ((n_edges + grain - 1) // grain) * grain
    pad = e_pad - n_edges
    src = jnp.concatenate([edge_index[0], jnp.zeros((pad,), jnp.int32)])
    dst = jnp.concatenate([edge_index[1], jnp.zeros((pad,), jnp.int32)])

    GA, GB = _gather_rows(A, B, src, dst, e_pad)
    scores = _edge_scores(GA, GB, W2, b2, e_pad)
    return scores[:n_edges]
